# baseline (speedup 1.0000x reference)
import functools

import jax
import jax.numpy as jnp
from jax import lax
from jax.experimental import pallas as pl
from jax.experimental.pallas import tpu as pltpu
from jax.experimental.pallas import tpu_sc as plsc

N = 10000
E = 320000
DF = 128
H = 64
T = 4

NC = 2
NS = 16
NW = NC * NS
EPT = E // NW
C = 96
CH = EPT // C
CT = EPT - CH * C
NA = 10240
RPT = NA // NS

_EPS = 1e-5


def _bn_relu(y, g, b):
    mean = jnp.mean(y, axis=0, keepdims=True)
    var = jnp.mean((y - mean) ** 2, axis=0, keepdims=True)
    return jnp.maximum((y - mean) / jnp.sqrt(var + _EPS) * g + b, 0.0)


def _head_body(x_ref, w_ref, b_ref, g_ref, be_ref, out_ref):
    y = jnp.dot(x_ref[...], w_ref[...], preferred_element_type=jnp.float32)
    h = _bn_relu(y + b_ref[...], g_ref[...], be_ref[...])
    out_ref[...] = jnp.concatenate(
        [h, jnp.zeros((NA - N, H), jnp.float32)], axis=0)


def _tail_body(h_ref, p_ref, gw_ref, gb_ref, gg_ref, gbe_ref,
               lw_ref, lb_ref, tw_ref, tb_ref, out_ref):
    z = h_ref[0:N, :] + p_ref[0:N, 0:H] + p_ref[0:N, H:2 * H]
    y = jnp.dot(z, gw_ref[...], preferred_element_type=jnp.float32)
    h2 = _bn_relu(y + gb_ref[...], gg_ref[...], gbe_ref[...])
    w2 = jnp.dot(lw_ref[...], tw_ref[...], preferred_element_type=jnp.float32)
    b2 = jnp.dot(lb_ref[...], tw_ref[...],
                 preferred_element_type=jnp.float32) + tb_ref[...]
    out_ref[...] = jnp.dot(h2, w2, preferred_element_type=jnp.float32) + b2


LCH = (CH // 4) * 4


def _seg_sum_body(h_hbm, ei_hbm, zeros_hbm, out_hbm,
                  src_v, dst_v, rows0, rows1, rows2, rows3, rows_t,
                  aggr_s, h_s,
                  sg0, sg1, sg2, sg3, ss0, ss1, ss2, ss3):
    cid = lax.axis_index("c")
    sid = lax.axis_index("s")
    wid = cid * NS + sid
    base = wid * EPT
    sg = (sg0, sg1, sg2, sg3)
    ss = (ss0, ss1, ss2, ss3)
    rows = (rows0, rows1, rows2, rows3)
    gsrc = (h_hbm, h_s, h_hbm, h_s)

    def gather_desc(c, b):
        return pltpu.make_async_copy(
            gsrc[b].at[src_v.at[pl.ds(c * C, C)]], rows[b], sg[b])

    def scatter_desc(c, b):
        return pltpu.make_async_copy(
            rows[b], aggr_s.at[dst_v.at[pl.ds(c * C, C)]], ss[b])

    def scatter_start(c, b):
        pltpu.async_copy(rows[b], aggr_s.at[dst_v.at[pl.ds(c * C, C)]],
                         ss[b], add=True)

    pltpu.sync_copy(zeros_hbm.at[pl.ds(sid * RPT, RPT)],
                    aggr_s.at[pl.ds(sid * RPT, RPT)])
    pltpu.sync_copy(h_hbm.at[pl.ds(sid * RPT, RPT)],
                    h_s.at[pl.ds(sid * RPT, RPT)])
    pltpu.sync_copy(ei_hbm.at[pl.ds(base, EPT)], src_v)
    pltpu.sync_copy(ei_hbm.at[pl.ds(E + base, EPT)], dst_v)
    plsc.subcore_barrier()
    gather_desc(0, 0).start()
    gather_desc(1, 1).start()

    @pl.loop(0, LCH, step=4)
    def _(j):
        for b in range(4):
            c = j + b

            @pl.when(c >= 2)
            def _():
                scatter_desc(c - 2, (b + 2) % 4).wait()

            @pl.when(c + 2 < CH)
            def _():
                gather_desc(c + 2, (b + 2) % 4).start()

            gather_desc(c, b).wait()
            scatter_start(c, b)

    for c in range(LCH, CH):
        scatter_desc(c - 2, (c - 2) % 4).wait()
        gather_desc(c, c % 4).wait()
        scatter_start(c, c % 4)
    for c in range(CH - 2, CH):
        scatter_desc(c, c % 4).wait()
    pltpu.async_copy(h_s.at[src_v.at[pl.ds(CH * C, CT)]], rows_t,
                     sg0).wait()
    pltpu.sync_copy(rows_t, aggr_s.at[dst_v.at[pl.ds(CH * C, CT)]], add=True)

    plsc.subcore_barrier()
    pltpu.sync_copy(aggr_s.at[pl.ds(sid * RPT, RPT)],
                    out_hbm.at[pl.ds(sid * RPT, RPT), pl.ds(cid * H, H)])


_seg_sum = functools.partial(
    pl.kernel,
    out_type=jax.ShapeDtypeStruct((NA, NC * H), jnp.float32),
    mesh=plsc.VectorSubcoreMesh(core_axis_name="c", subcore_axis_name="s"),
    scratch_types=[
        pltpu.VMEM((EPT,), jnp.int32),
        pltpu.VMEM((EPT,), jnp.int32),
        pltpu.VMEM((C, H), jnp.float32),
        pltpu.VMEM((C, H), jnp.float32),
        pltpu.VMEM((C, H), jnp.float32),
        pltpu.VMEM((C, H), jnp.float32),
        pltpu.VMEM((CT, H), jnp.float32),
        pltpu.VMEM_SHARED((NA, H), jnp.float32),
        pltpu.VMEM_SHARED((NA, H), jnp.float32),
        pltpu.SemaphoreType.DMA,
        pltpu.SemaphoreType.DMA,
        pltpu.SemaphoreType.DMA,
        pltpu.SemaphoreType.DMA,
        pltpu.SemaphoreType.DMA,
        pltpu.SemaphoreType.DMA,
        pltpu.SemaphoreType.DMA,
        pltpu.SemaphoreType.DMA,
    ],
    compiler_params=pltpu.CompilerParams(use_tc_tiling_on_sc=False),
)(_seg_sum_body)


def kernel(x, edge_index, batch, head_W, head_b, head_g, head_be,
           gin_W, gin_b, gin_g, gin_be, lin_W, lin_b, tail_W, tail_b):
    del batch
    h_pad = pl.pallas_call(
        _head_body,
        out_shape=jax.ShapeDtypeStruct((NA, H), jnp.float32),
    )(x, head_W, head_b.reshape(1, H), head_g.reshape(1, H),
      head_be.reshape(1, H))

    zeros = jnp.zeros((NA, H), jnp.float32)
    parts = _seg_sum(h_pad, edge_index.reshape(2 * E), zeros)

    out = pl.pallas_call(
        _tail_body,
        out_shape=jax.ShapeDtypeStruct((N, T), jnp.float32),
    )(h_pad, parts, gin_W, gin_b.reshape(1, H), gin_g.reshape(1, H),
      gin_be.reshape(1, H), lin_W, lin_b.reshape(1, H), tail_W,
      tail_b.reshape(1, T))
    return out


# speedup vs baseline: 1.1198x; 1.1198x over previous
import functools

import jax
import jax.numpy as jnp
from jax import lax
from jax.experimental import pallas as pl
from jax.experimental.pallas import tpu as pltpu
from jax.experimental.pallas import tpu_sc as plsc

N = 10000
E = 320000
DF = 128
H = 64
T = 4

NC = 2
NS = 16
NW = NC * NS
EPT = E // NW
C = 96
CH = EPT // C
CT = EPT - CH * C
NA = 10240
RPT = NA // NS

_EPS = 1e-5


def _bn_relu(y, g, b):
    mean = jnp.mean(y, axis=0, keepdims=True)
    var = jnp.mean((y - mean) ** 2, axis=0, keepdims=True)
    return jnp.maximum((y - mean) / jnp.sqrt(var + _EPS) * g + b, 0.0)


def _head_body(x_ref, w_ref, b_ref, g_ref, be_ref, out_ref):
    y = jnp.dot(x_ref[...], w_ref[...], preferred_element_type=jnp.float32)
    h = _bn_relu(y + b_ref[...], g_ref[...], be_ref[...])
    out_ref[...] = jnp.concatenate(
        [h, jnp.zeros((NA - N, H), jnp.float32)], axis=0)


def _tail_body(h_ref, p_ref, gw_ref, gb_ref, gg_ref, gbe_ref,
               lw_ref, lb_ref, tw_ref, tb_ref, out_ref):
    z = h_ref[0:N, :] + p_ref[0:N, 0:H] + p_ref[0:N, H:2 * H]
    y = jnp.dot(z, gw_ref[...], preferred_element_type=jnp.float32)
    h2 = _bn_relu(y + gb_ref[...], gg_ref[...], gbe_ref[...])
    w2 = jnp.dot(lw_ref[...], tw_ref[...], preferred_element_type=jnp.float32)
    b2 = jnp.dot(lb_ref[...], tw_ref[...],
                 preferred_element_type=jnp.float32) + tb_ref[...]
    out_ref[...] = jnp.dot(h2, w2, preferred_element_type=jnp.float32) + b2


LCH = (CH // 4) * 4


def _seg_sum_body(h_hbm, ei_hbm, zeros_hbm, out_hbm,
                  src_v, dst_v, rows0, rows1, rows2, rows3, rows_t,
                  aggr_s, h_s,
                  sg0, sg1, sg2, sg3, ss0, ss1, ss2, ss3):
    cid = lax.axis_index("c")
    sid = lax.axis_index("s")
    wid = cid * NS + sid
    base = wid * EPT
    sg = (sg0, sg1, sg2, sg3)
    ss = (ss0, ss1, ss2, ss3)
    rows = (rows0, rows1, rows2, rows3)
    gsrc = (h_s, h_s, h_s, h_s)

    def gather_desc(c, b):
        return pltpu.make_async_copy(
            gsrc[b].at[src_v.at[pl.ds(c * C, C)]], rows[b], sg[b])

    def scatter_desc(c, b):
        return pltpu.make_async_copy(
            rows[b], aggr_s.at[dst_v.at[pl.ds(c * C, C)]], ss[b])

    def scatter_start(c, b):
        pltpu.async_copy(rows[b], aggr_s.at[dst_v.at[pl.ds(c * C, C)]],
                         ss[b], add=True)

    pltpu.sync_copy(zeros_hbm.at[pl.ds(sid * RPT, RPT)],
                    aggr_s.at[pl.ds(sid * RPT, RPT)])
    pltpu.sync_copy(h_hbm.at[pl.ds(sid * RPT, RPT)],
                    h_s.at[pl.ds(sid * RPT, RPT)])
    pltpu.sync_copy(ei_hbm.at[pl.ds(base, EPT)], src_v)
    pltpu.sync_copy(ei_hbm.at[pl.ds(E + base, EPT)], dst_v)
    plsc.subcore_barrier()
    gather_desc(0, 0).start()
    gather_desc(1, 1).start()

    @pl.loop(0, LCH, step=4)
    def _(j):
        for b in range(4):
            c = j + b

            @pl.when(c >= 2)
            def _():
                scatter_desc(c - 2, (b + 2) % 4).wait()

            @pl.when(c + 2 < CH)
            def _():
                gather_desc(c + 2, (b + 2) % 4).start()

            gather_desc(c, b).wait()
            scatter_start(c, b)

    for c in range(LCH, CH):
        scatter_desc(c - 2, (c - 2) % 4).wait()
        gather_desc(c, c % 4).wait()
        scatter_start(c, c % 4)
    for c in range(CH - 2, CH):
        scatter_desc(c, c % 4).wait()
    pltpu.async_copy(h_s.at[src_v.at[pl.ds(CH * C, CT)]], rows_t,
                     sg0).wait()
    pltpu.sync_copy(rows_t, aggr_s.at[dst_v.at[pl.ds(CH * C, CT)]], add=True)

    plsc.subcore_barrier()
    pltpu.sync_copy(aggr_s.at[pl.ds(sid * RPT, RPT)],
                    out_hbm.at[pl.ds(sid * RPT, RPT), pl.ds(cid * H, H)])


_seg_sum = functools.partial(
    pl.kernel,
    out_type=jax.ShapeDtypeStruct((NA, NC * H), jnp.float32),
    mesh=plsc.VectorSubcoreMesh(core_axis_name="c", subcore_axis_name="s"),
    scratch_types=[
        pltpu.VMEM((EPT,), jnp.int32),
        pltpu.VMEM((EPT,), jnp.int32),
        pltpu.VMEM((C, H), jnp.float32),
        pltpu.VMEM((C, H), jnp.float32),
        pltpu.VMEM((C, H), jnp.float32),
        pltpu.VMEM((C, H), jnp.float32),
        pltpu.VMEM((CT, H), jnp.float32),
        pltpu.VMEM_SHARED((NA, H), jnp.float32),
        pltpu.VMEM_SHARED((NA, H), jnp.float32),
        pltpu.SemaphoreType.DMA,
        pltpu.SemaphoreType.DMA,
        pltpu.SemaphoreType.DMA,
        pltpu.SemaphoreType.DMA,
        pltpu.SemaphoreType.DMA,
        pltpu.SemaphoreType.DMA,
        pltpu.SemaphoreType.DMA,
        pltpu.SemaphoreType.DMA,
    ],
    compiler_params=pltpu.CompilerParams(use_tc_tiling_on_sc=False),
)(_seg_sum_body)


def kernel(x, edge_index, batch, head_W, head_b, head_g, head_be,
           gin_W, gin_b, gin_g, gin_be, lin_W, lin_b, tail_W, tail_b):
    del batch
    h_pad = pl.pallas_call(
        _head_body,
        out_shape=jax.ShapeDtypeStruct((NA, H), jnp.float32),
    )(x, head_W, head_b.reshape(1, H), head_g.reshape(1, H),
      head_be.reshape(1, H))

    zeros = jnp.zeros((NA, H), jnp.float32)
    parts = _seg_sum(h_pad, edge_index.reshape(2 * E), zeros)

    out = pl.pallas_call(
        _tail_body,
        out_shape=jax.ShapeDtypeStruct((N, T), jnp.float32),
    )(h_pad, parts, gin_W, gin_b.reshape(1, H), gin_g.reshape(1, H),
      gin_be.reshape(1, H), lin_W, lin_b.reshape(1, H), tail_W,
      tail_b.reshape(1, T))
    return out
